# baseline (speedup 1.0000x reference)
import functools

import jax
import jax.numpy as jnp
from jax import lax
from jax.experimental import pallas as pl
from jax.experimental.pallas import tpu as pltpu
from jax.experimental.pallas import tpu_sc as plsc

DIM = 2048
N_CHANNELS = 32
TABLE = 2 * DIM - 1
PADS = 16
WIDTH = TABLE + PADS + 1
N_Q = 8
N_K = 8
N_V = 16
K_INFLIGHT = 8


def _emit_rows(dst_hbm, c_local, buf, sem):
    nblk = DIM // PADS

    pltpu.make_async_copy(
        buf.at[pl.ds(1, PADS - 1), pl.ds(DIM, DIM)],
        dst_hbm.at[c_local, pl.ds(0, PADS - 1)],
        sem,
    ).start()
    pltpu.make_async_copy(
        buf.at[0, pl.ds(0, DIM)], dst_hbm.at[c_local, DIM - 1], sem
    ).start()

    def body(j, carry):
        a = pl.multiple_of(DIM - PADS * j, PADS)
        pltpu.make_async_copy(
            buf.at[:, pl.ds(a, DIM)],
            dst_hbm.at[c_local, pl.ds(PADS * j - 1, PADS)],
            sem,
        ).start()

        @pl.when(j >= 1 + K_INFLIGHT)
        def _():
            pltpu.make_async_copy(
                buf.at[:, pl.ds(0, DIM)],
                dst_hbm.at[c_local, pl.ds(0, PADS)],
                sem,
            ).wait()

        return carry

    lax.fori_loop(1, nblk, body, 0)

    def drain(i, carry):
        pltpu.make_async_copy(
            buf.at[:, pl.ds(0, DIM)], dst_hbm.at[c_local, pl.ds(0, PADS)], sem
        ).wait()
        return carry

    lax.fori_loop(0, K_INFLIGHT, drain, 0)
    pltpu.make_async_copy(
        buf.at[pl.ds(1, PADS - 1), pl.ds(DIM, DIM)],
        dst_hbm.at[c_local, pl.ds(0, PADS - 1)],
        sem,
    ).wait()
    pltpu.make_async_copy(
        buf.at[0, pl.ds(0, DIM)], dst_hbm.at[c_local, DIM - 1], sem
    ).wait()


@functools.partial(
    pl.kernel,
    out_type=(
        jax.ShapeDtypeStruct((N_Q, DIM, DIM), jnp.float32),
        jax.ShapeDtypeStruct((N_K, DIM, DIM), jnp.float32),
        jax.ShapeDtypeStruct((N_V, DIM, DIM), jnp.float32),
    ),
    mesh=plsc.VectorSubcoreMesh(core_axis_name="c", subcore_axis_name="s"),
    compiler_params=pltpu.CompilerParams(use_tc_tiling_on_sc=False),
    scratch_types=[
        pltpu.VMEM((PADS, WIDTH), jnp.float32),
        pltpu.SemaphoreType.DMA,
    ],
)
def _sc_expand(shifted_hbm, q_hbm, k_hbm, v_hbm, buf, sem):
    wid = lax.axis_index("s") * 2 + lax.axis_index("c")
    pltpu.sync_copy(shifted_hbm.at[wid], buf)

    @pl.when(wid < N_Q)
    def _():
        _emit_rows(q_hbm, wid, buf, sem)

    @pl.when((wid >= N_Q) & (wid < N_Q + N_K))
    def _():
        _emit_rows(k_hbm, wid - N_Q, buf, sem)

    @pl.when(wid >= N_Q + N_K)
    def _():
        _emit_rows(v_hbm, wid - (N_Q + N_K), buf, sem)


def kernel(relative, flatten_index):
    del flatten_index
    rev = relative[:, ::-1]
    shifted = jnp.stack(
        [jnp.pad(rev, ((0, 0), (p, PADS + 1 - p))) for p in range(PADS)],
        axis=1,
    )
    return _sc_expand(shifted)
